# baseline (speedup 1.0000x reference)
import functools

import jax
import jax.numpy as jnp
from jax import lax
from jax.experimental import pallas as pl
from jax.experimental.pallas import tpu as pltpu
from jax.experimental.pallas import tpu_sc as plsc

N = 10000
HID = 16
NC, NS = 2, 16
NW = NC * NS
CB = 128
NPAD = 10240
RPT = NPAD // NS
NBUF = 2

_mesh = plsc.VectorSubcoreMesh(core_axis_name="c", subcore_axis_name="s")


def _fill(buf, val, rows):
    def body(i, _):
        buf[i, :] = jnp.full((HID,), val, jnp.float32)
        return 0
    lax.fori_loop(0, rows, body, 0)


def _chunk_split(tch):
    return tch // NW, tch % NW


def _make_deg_kernel(tch):
    bpt, rem = _chunk_split(tch)

    @functools.partial(
        pl.kernel,
        mesh=_mesh,
        out_type=jax.ShapeDtypeStruct((NC, NPAD, HID), jnp.float32),
        scratch_types=[
            pltpu.VMEM((bpt + 1, CB), jnp.int32),
            pltpu.VMEM((CB, HID), jnp.float32),
            pltpu.VMEM_SHARED((NPAD, HID), jnp.float32),
        ],
        compiler_params=pltpu.CompilerParams(use_tc_tiling_on_sc=False),
    )
    def deg_kernel(dst_hbm, out_hbm, dst_v, buf_v, acc_sh):
        c = lax.axis_index("c")
        s = lax.axis_index("s")
        wid = c * NS + s
        cw = bpt * wid + jnp.minimum(wid, rem)
        _fill(buf_v, 0.0, CB)
        for k in range(RPT // CB):
            pltpu.sync_copy(buf_v, acc_sh.at[pl.ds(s * RPT + k * CB, CB)])
        _fill(buf_v, 1.0, CB)
        pltpu.sync_copy(dst_hbm.at[pl.ds(cw, bpt)],
                        dst_v.at[pl.ds(0, bpt)])

        @pl.when(wid < rem)
        def _():
            pltpu.sync_copy(dst_hbm.at[pl.ds(cw + bpt, 1)],
                            dst_v.at[pl.ds(bpt, 1)])
        plsc.subcore_barrier()

        def body(j, _):
            pltpu.sync_copy(buf_v, acc_sh.at[dst_v.at[j]], add=True)
            return 0
        lax.fori_loop(0, bpt, body, 0)

        @pl.when(wid < rem)
        def _():
            pltpu.sync_copy(buf_v, acc_sh.at[dst_v.at[bpt]], add=True)
        plsc.subcore_barrier()
        pltpu.sync_copy(acc_sh.at[pl.ds(s * RPT, RPT)],
                        out_hbm.at[c, pl.ds(s * RPT, RPT)])

    return deg_kernel


def _make_agg_kernel(tch):
    bpt, rem = _chunk_split(tch)
    assert bpt % NBUF == 0

    @functools.partial(
        pl.kernel,
        mesh=_mesh,
        out_type=jax.ShapeDtypeStruct((NC, NPAD, HID), jnp.float32),
        scratch_types=[
            pltpu.VMEM((bpt + 1, CB), jnp.int32),
            pltpu.VMEM((bpt + 1, CB), jnp.int32),
            [pltpu.VMEM((CB, HID), jnp.float32)] * NBUF,
            pltpu.VMEM_SHARED((NPAD, HID), jnp.float32),
            pltpu.VMEM_SHARED((NPAD, HID), jnp.float32),
            [pltpu.SemaphoreType.DMA] * NBUF,
        ],
        compiler_params=pltpu.CompilerParams(use_tc_tiling_on_sc=False),
    )
    def agg_kernel(g_hbm, src_hbm, dst_hbm, out_hbm,
                   src_v, dst_v, bufs, acc_sh, g_sh, sems):
        c = lax.axis_index("c")
        s = lax.axis_index("s")
        wid = c * NS + s
        cw = bpt * wid + jnp.minimum(wid, rem)
        _fill(bufs[0], 0.0, CB)
        for k in range(RPT // CB):
            pltpu.sync_copy(bufs[0], acc_sh.at[pl.ds(s * RPT + k * CB, CB)])
        pltpu.sync_copy(g_hbm.at[pl.ds(s * RPT, RPT)],
                        g_sh.at[pl.ds(s * RPT, RPT)])
        pltpu.sync_copy(src_hbm.at[pl.ds(cw, bpt)], src_v.at[pl.ds(0, bpt)])
        pltpu.sync_copy(dst_hbm.at[pl.ds(cw, bpt)], dst_v.at[pl.ds(0, bpt)])

        @pl.when(wid < rem)
        def _():
            pltpu.sync_copy(src_hbm.at[pl.ds(cw + bpt, 1)],
                            src_v.at[pl.ds(bpt, 1)])
            pltpu.sync_copy(dst_hbm.at[pl.ds(cw + bpt, 1)],
                            dst_v.at[pl.ds(bpt, 1)])
        plsc.subcore_barrier()

        for b in range(NBUF):
            pltpu.async_copy(g_sh.at[src_v.at[b]], bufs[b], sems[b])

        def round_body(g, _):
            for b in range(NBUF):
                j = g * NBUF + b
                pltpu.make_async_copy(
                    g_hbm.at[pl.ds(0, CB)], bufs[b], sems[b]).wait()
                pltpu.sync_copy(bufs[b], acc_sh.at[dst_v.at[j]], add=True)
                pltpu.async_copy(
                    g_sh.at[src_v.at[j + NBUF]], bufs[b], sems[b])
            return 0
        lax.fori_loop(0, bpt // NBUF - 1, round_body, 0)
        for b in range(NBUF):
            j = bpt - NBUF + b
            pltpu.make_async_copy(
                g_hbm.at[pl.ds(0, CB)], bufs[b], sems[b]).wait()
            pltpu.sync_copy(bufs[b], acc_sh.at[dst_v.at[j]], add=True)

        @pl.when(wid < rem)
        def _():
            pltpu.async_copy(g_sh.at[src_v.at[bpt]], bufs[0], sems[0]).wait()
            pltpu.sync_copy(bufs[0], acc_sh.at[dst_v.at[bpt]], add=True)
        plsc.subcore_barrier()
        pltpu.sync_copy(acc_sh.at[pl.ds(s * RPT, RPT)],
                        out_hbm.at[c, pl.ds(s * RPT, RPT)])

    return agg_kernel


def _dense1_body(deg_ref, x_ref, w1_ref, dinv_ref, g1_ref):
    deg16 = deg_ref[0] + deg_ref[1]
    dinv = lax.rsqrt(deg16 + 1.0)
    h = jnp.dot(x_ref[...], w1_ref[...], preferred_element_type=jnp.float32)
    dinv_ref[...] = dinv
    g1_ref[pl.ds(0, N), :] = dinv[:N, :] * h
    g1_ref[pl.ds(N, NPAD - N), :] = jnp.zeros((NPAD - N, HID), jnp.float32)


def _dense2_body(a_ref, g_ref, dinv_ref, b_ref, w2_ref, g2_ref):
    acc = a_ref[0] + a_ref[1] + g_ref[...]
    h1 = jnp.maximum(dinv_ref[...] * acc + b_ref[...], 0.0)
    g2_ref[...] = dinv_ref[...] * jnp.dot(
        h1, w2_ref[...], preferred_element_type=jnp.float32)


def _final_body(a_ref, g_ref, dinv_ref, b_ref, wfc_ref, bfc_ref, out_ref):
    acc = a_ref[0] + a_ref[1] + g_ref[...]
    h2 = jnp.maximum(dinv_ref[...] * acc + b_ref[...], 0.0)
    mask = lax.broadcasted_iota(jnp.int32, (NPAD, HID), 0) < N
    h2 = jnp.where(mask, h2, 0.0)
    pooled = jnp.sum(h2, axis=0, keepdims=True) * (1.0 / N)
    y = jnp.sum(pooled * wfc_ref[...], axis=1, keepdims=True) + bfc_ref[...]
    out_ref[...] = 1.0 / (1.0 + jnp.exp(-y))


def kernel(x, edge_index, W1, b1, W2, b2, Wfc, bfc):
    E = edge_index.shape[1]
    tch = E // CB
    src2d = edge_index[0].astype(jnp.int32).reshape(tch, CB)
    dst2d = edge_index[1].astype(jnp.int32).reshape(tch, CB)

    deg_parts = _make_deg_kernel(tch)(dst2d)

    dinv16, g1 = pl.pallas_call(
        _dense1_body,
        out_shape=[jax.ShapeDtypeStruct((NPAD, HID), jnp.float32),
                   jax.ShapeDtypeStruct((NPAD, HID), jnp.float32)],
    )(deg_parts, x, W1)

    agg = _make_agg_kernel(tch)
    a1 = agg(g1, src2d, dst2d)

    g2 = pl.pallas_call(
        _dense2_body,
        out_shape=jax.ShapeDtypeStruct((NPAD, HID), jnp.float32),
    )(a1, g1, dinv16, b1.reshape(1, HID), W2)

    a2 = agg(g2, src2d, dst2d)

    y = pl.pallas_call(
        _final_body,
        out_shape=jax.ShapeDtypeStruct((1, 1), jnp.float32),
    )(a2, g2, dinv16, b2.reshape(1, HID), Wfc.reshape(1, HID),
      bfc.reshape(1, 1))
    return y.reshape(1)


# speedup vs baseline: 1.0602x; 1.0602x over previous
import functools

import jax
import jax.numpy as jnp
from jax import lax
from jax.experimental import pallas as pl
from jax.experimental.pallas import tpu as pltpu
from jax.experimental.pallas import tpu_sc as plsc

N = 10000
HID = 16
NC, NS = 2, 16
NW = NC * NS
CB = 128
NPAD = 10240
RPT = NPAD // NS
NBUF = 4

_mesh = plsc.VectorSubcoreMesh(core_axis_name="c", subcore_axis_name="s")


def _fill(buf, val, rows):
    def body(i, _):
        buf[i, :] = jnp.full((HID,), val, jnp.float32)
        return 0
    lax.fori_loop(0, rows, body, 0)


def _chunk_split(tch):
    return tch // NW, tch % NW


def _make_deg_kernel(tch):
    bpt, rem = _chunk_split(tch)

    @functools.partial(
        pl.kernel,
        mesh=_mesh,
        out_type=jax.ShapeDtypeStruct((NC, NPAD, HID), jnp.float32),
        scratch_types=[
            pltpu.VMEM((bpt + 1, CB), jnp.int32),
            pltpu.VMEM((CB, HID), jnp.float32),
            pltpu.VMEM_SHARED((NPAD, HID), jnp.float32),
            pltpu.SemaphoreType.DMA,
        ],
        compiler_params=pltpu.CompilerParams(use_tc_tiling_on_sc=False),
    )
    def deg_kernel(dst_hbm, out_hbm, dst_v, buf_v, acc_sh, sem):
        c = lax.axis_index("c")
        s = lax.axis_index("s")
        wid = c * NS + s
        cw = bpt * wid + jnp.minimum(wid, rem)
        _fill(buf_v, 0.0, CB)
        for k in range(RPT // CB):
            pltpu.sync_copy(buf_v, acc_sh.at[pl.ds(s * RPT + k * CB, CB)])
        _fill(buf_v, 1.0, CB)
        pltpu.sync_copy(dst_hbm.at[pl.ds(cw, bpt)],
                        dst_v.at[pl.ds(0, bpt)])

        @pl.when(wid < rem)
        def _():
            pltpu.sync_copy(dst_hbm.at[pl.ds(cw + bpt, 1)],
                            dst_v.at[pl.ds(bpt, 1)])
        plsc.subcore_barrier()

        depth = 6
        for j in range(depth):
            pltpu.async_copy(buf_v, acc_sh.at[dst_v.at[j]], sem, add=True)

        def body(j, _):
            pltpu.make_async_copy(out_hbm.at[0, pl.ds(0, CB)], buf_v,
                                  sem).wait()
            pltpu.async_copy(buf_v, acc_sh.at[dst_v.at[j + depth]], sem,
                             add=True)
            return 0
        lax.fori_loop(0, bpt - depth, body, 0)

        @pl.when(wid < rem)
        def _():
            pltpu.async_copy(buf_v, acc_sh.at[dst_v.at[bpt]], sem, add=True)
        for _ in range(depth):
            pltpu.make_async_copy(out_hbm.at[0, pl.ds(0, CB)], buf_v,
                                  sem).wait()

        @pl.when(wid < rem)
        def _():
            pltpu.make_async_copy(out_hbm.at[0, pl.ds(0, CB)], buf_v,
                                  sem).wait()
        plsc.subcore_barrier()
        pltpu.sync_copy(acc_sh.at[pl.ds(s * RPT, RPT)],
                        out_hbm.at[c, pl.ds(s * RPT, RPT)])

    return deg_kernel


def _make_agg_kernel(tch):
    bpt, rem = _chunk_split(tch)

    @functools.partial(
        pl.kernel,
        mesh=_mesh,
        out_type=jax.ShapeDtypeStruct((NC, NPAD, HID), jnp.float32),
        scratch_types=[
            pltpu.VMEM((bpt + 1, CB), jnp.int32),
            pltpu.VMEM((bpt + 1, CB), jnp.int32),
            [pltpu.VMEM((CB, HID), jnp.float32)] * NBUF,
            pltpu.VMEM_SHARED((NPAD, HID), jnp.float32),
            pltpu.VMEM_SHARED((NPAD, HID), jnp.float32),
            [pltpu.SemaphoreType.DMA] * NBUF,
        ],
        compiler_params=pltpu.CompilerParams(use_tc_tiling_on_sc=False),
    )
    def agg_kernel(g_hbm, src_hbm, dst_hbm, out_hbm,
                   src_v, dst_v, bufs, acc_sh, g_sh, sems):
        c = lax.axis_index("c")
        s = lax.axis_index("s")
        wid = c * NS + s
        cw = bpt * wid + jnp.minimum(wid, rem)
        _fill(bufs[0], 0.0, CB)
        for k in range(RPT // CB):
            pltpu.sync_copy(bufs[0], acc_sh.at[pl.ds(s * RPT + k * CB, CB)])
        pltpu.sync_copy(g_hbm.at[pl.ds(s * RPT, RPT)],
                        g_sh.at[pl.ds(s * RPT, RPT)])
        pltpu.sync_copy(src_hbm.at[pl.ds(cw, bpt)], src_v.at[pl.ds(0, bpt)])
        pltpu.sync_copy(dst_hbm.at[pl.ds(cw, bpt)], dst_v.at[pl.ds(0, bpt)])

        @pl.when(wid < rem)
        def _():
            pltpu.sync_copy(src_hbm.at[pl.ds(cw + bpt, 1)],
                            src_v.at[pl.ds(bpt, 1)])
            pltpu.sync_copy(dst_hbm.at[pl.ds(cw + bpt, 1)],
                            dst_v.at[pl.ds(bpt, 1)])
        plsc.subcore_barrier()

        def drain(b):
            pltpu.make_async_copy(g_hbm.at[pl.ds(0, CB)], bufs[b],
                                  sems[b]).wait()

        def emit_visit(j_val, b, bp, with_reuse_wait):
            if with_reuse_wait:
                drain(b)
            pltpu.async_copy(g_sh.at[src_v.at[j_val]], bufs[b], sems[b])
            drain(bp)
            pltpu.async_copy(bufs[bp], acc_sh.at[dst_v.at[j_val - 1]],
                             sems[bp], add=True)

        pltpu.async_copy(g_sh.at[src_v.at[0]], bufs[0], sems[0])
        peel_extra = (bpt - NBUF) % NBUF
        n_rounds = (bpt - NBUF - peel_extra) // NBUF
        for j in range(1, NBUF + peel_extra):
            emit_visit(j, j % NBUF, (j - 1) % NBUF, j >= NBUF)

        j0 = NBUF + peel_extra

        def round_body(g, _):
            base = j0 + g * NBUF
            for u in range(NBUF):
                b = (j0 + u) % NBUF
                bp = (j0 + u - 1) % NBUF
                emit_visit(base + u, b, bp, True)
            return 0
        lax.fori_loop(0, n_rounds, round_body, 0)
        drain((bpt - 1) % NBUF)
        pltpu.async_copy(bufs[(bpt - 1) % NBUF],
                         acc_sh.at[dst_v.at[bpt - 1]],
                         sems[(bpt - 1) % NBUF], add=True)

        for b in range(NBUF):
            drain(b)

        @pl.when(wid < rem)
        def _():
            pltpu.async_copy(g_sh.at[src_v.at[bpt]], bufs[0], sems[0]).wait()
            pltpu.async_copy(bufs[0], acc_sh.at[dst_v.at[bpt]],
                             sems[0], add=True)
            drain(0)
        plsc.subcore_barrier()
        pltpu.sync_copy(acc_sh.at[pl.ds(s * RPT, RPT)],
                        out_hbm.at[c, pl.ds(s * RPT, RPT)])

    return agg_kernel


def _dense1_body(deg_ref, x_ref, w1_ref, dinv_ref, g1_ref):
    deg16 = deg_ref[0] + deg_ref[1]
    dinv = lax.rsqrt(deg16 + 1.0)
    h = jnp.dot(x_ref[...], w1_ref[...], preferred_element_type=jnp.float32)
    dinv_ref[...] = dinv
    g1_ref[pl.ds(0, N), :] = dinv[:N, :] * h
    g1_ref[pl.ds(N, NPAD - N), :] = jnp.zeros((NPAD - N, HID), jnp.float32)


def _dense2_body(a_ref, g_ref, dinv_ref, b_ref, w2_ref, g2_ref):
    acc = a_ref[0] + a_ref[1] + g_ref[...]
    h1 = jnp.maximum(dinv_ref[...] * acc + b_ref[...], 0.0)
    g2_ref[...] = dinv_ref[...] * jnp.dot(
        h1, w2_ref[...], preferred_element_type=jnp.float32)


def _final_body(a_ref, g_ref, dinv_ref, b_ref, wfc_ref, bfc_ref, out_ref):
    acc = a_ref[0] + a_ref[1] + g_ref[...]
    h2 = jnp.maximum(dinv_ref[...] * acc + b_ref[...], 0.0)
    mask = lax.broadcasted_iota(jnp.int32, (NPAD, HID), 0) < N
    h2 = jnp.where(mask, h2, 0.0)
    pooled = jnp.sum(h2, axis=0, keepdims=True) * (1.0 / N)
    y = jnp.sum(pooled * wfc_ref[...], axis=1, keepdims=True) + bfc_ref[...]
    out_ref[...] = 1.0 / (1.0 + jnp.exp(-y))


def kernel(x, edge_index, W1, b1, W2, b2, Wfc, bfc):
    E = edge_index.shape[1]
    tch = E // CB
    src2d = edge_index[0].astype(jnp.int32).reshape(tch, CB)
    dst2d = edge_index[1].astype(jnp.int32).reshape(tch, CB)

    deg_parts = _make_deg_kernel(tch)(dst2d)

    dinv16, g1 = pl.pallas_call(
        _dense1_body,
        out_shape=[jax.ShapeDtypeStruct((NPAD, HID), jnp.float32),
                   jax.ShapeDtypeStruct((NPAD, HID), jnp.float32)],
    )(deg_parts, x, W1)

    agg = _make_agg_kernel(tch)
    a1 = agg(g1, src2d, dst2d)

    g2 = pl.pallas_call(
        _dense2_body,
        out_shape=jax.ShapeDtypeStruct((NPAD, HID), jnp.float32),
    )(a1, g1, dinv16, b1.reshape(1, HID), W2)

    a2 = agg(g2, src2d, dst2d)

    y = pl.pallas_call(
        _final_body,
        out_shape=jax.ShapeDtypeStruct((1, 1), jnp.float32),
    )(a2, g2, dinv16, b2.reshape(1, HID), Wfc.reshape(1, HID),
      bfc.reshape(1, 1))
    return y.reshape(1)


# speedup vs baseline: 1.0603x; 1.0001x over previous
import functools

import jax
import jax.numpy as jnp
from jax import lax
from jax.experimental import pallas as pl
from jax.experimental.pallas import tpu as pltpu
from jax.experimental.pallas import tpu_sc as plsc

N = 10000
HID = 16
NC, NS = 2, 16
NW = NC * NS
CB = 128
NPAD = 10240
RPT = NPAD // NS
NBUF = 4

_mesh = plsc.VectorSubcoreMesh(core_axis_name="c", subcore_axis_name="s")


def _fill(buf, val, rows):
    def body(i, _):
        buf[i, :] = jnp.full((HID,), val, jnp.float32)
        return 0
    lax.fori_loop(0, rows, body, 0)


def _chunk_split(tch):
    return tch // NW, tch % NW


def _make_deg_kernel(tch):
    bpt, rem = _chunk_split(tch)

    @functools.partial(
        pl.kernel,
        mesh=_mesh,
        out_type=jax.ShapeDtypeStruct((NC, NPAD, HID), jnp.float32),
        scratch_types=[
            pltpu.VMEM((bpt + 1, CB), jnp.int32),
            pltpu.VMEM((CB, HID), jnp.float32),
            pltpu.VMEM_SHARED((NPAD, HID), jnp.float32),
            pltpu.SemaphoreType.DMA,
        ],
        compiler_params=pltpu.CompilerParams(use_tc_tiling_on_sc=False),
    )
    def deg_kernel(dst_hbm, out_hbm, dst_v, buf_v, acc_sh, sem):
        c = lax.axis_index("c")
        s = lax.axis_index("s")
        wid = c * NS + s
        cw = bpt * wid + jnp.minimum(wid, rem)
        _fill(buf_v, 0.0, CB)
        for k in range(RPT // CB):
            pltpu.sync_copy(buf_v, acc_sh.at[pl.ds(s * RPT + k * CB, CB)])
        _fill(buf_v, 1.0, CB)
        pltpu.sync_copy(dst_hbm.at[pl.ds(cw, bpt)],
                        dst_v.at[pl.ds(0, bpt)])

        @pl.when(wid < rem)
        def _():
            pltpu.sync_copy(dst_hbm.at[pl.ds(cw + bpt, 1)],
                            dst_v.at[pl.ds(bpt, 1)])
        plsc.subcore_barrier()

        depth = 6
        for j in range(depth):
            pltpu.async_copy(buf_v, acc_sh.at[dst_v.at[j]], sem, add=True)

        def body(j, _):
            pltpu.make_async_copy(out_hbm.at[0, pl.ds(0, CB)], buf_v,
                                  sem).wait()
            pltpu.async_copy(buf_v, acc_sh.at[dst_v.at[j + depth]], sem,
                             add=True)
            return 0
        lax.fori_loop(0, bpt - depth, body, 0)

        @pl.when(wid < rem)
        def _():
            pltpu.async_copy(buf_v, acc_sh.at[dst_v.at[bpt]], sem, add=True)
        for _ in range(depth):
            pltpu.make_async_copy(out_hbm.at[0, pl.ds(0, CB)], buf_v,
                                  sem).wait()

        @pl.when(wid < rem)
        def _():
            pltpu.make_async_copy(out_hbm.at[0, pl.ds(0, CB)], buf_v,
                                  sem).wait()
        plsc.subcore_barrier()
        pltpu.sync_copy(acc_sh.at[pl.ds(s * RPT, RPT)],
                        out_hbm.at[c, pl.ds(s * RPT, RPT)])

    return deg_kernel


def _make_agg_kernel(tch):
    bpt, rem = _chunk_split(tch)

    @functools.partial(
        pl.kernel,
        mesh=_mesh,
        out_type=jax.ShapeDtypeStruct((NC, NPAD, HID), jnp.float32),
        scratch_types=[
            pltpu.VMEM((bpt + 1, CB), jnp.int32),
            pltpu.VMEM((bpt + 1, CB), jnp.int32),
            [pltpu.VMEM((CB, HID), jnp.float32)] * NBUF,
            pltpu.VMEM_SHARED((NPAD, HID), jnp.float32),
            pltpu.VMEM_SHARED((NPAD, HID), jnp.float32),
            [pltpu.SemaphoreType.DMA] * NBUF,
        ],
        compiler_params=pltpu.CompilerParams(use_tc_tiling_on_sc=False),
    )
    def agg_kernel(g_hbm, src_hbm, dst_hbm, out_hbm,
                   src_v, dst_v, bufs, acc_sh, g_sh, sems):
        c = lax.axis_index("c")
        s = lax.axis_index("s")
        wid = c * NS + s
        cw = bpt * wid + jnp.minimum(wid, rem)
        _fill(bufs[0], 0.0, CB)
        for k in range(RPT // CB):
            pltpu.sync_copy(bufs[0], acc_sh.at[pl.ds(s * RPT + k * CB, CB)])
        pltpu.sync_copy(g_hbm.at[pl.ds(s * RPT, RPT)],
                        g_sh.at[pl.ds(s * RPT, RPT)])
        pltpu.sync_copy(src_hbm.at[pl.ds(cw, bpt)], src_v.at[pl.ds(0, bpt)])
        pltpu.sync_copy(dst_hbm.at[pl.ds(cw, bpt)], dst_v.at[pl.ds(0, bpt)])

        @pl.when(wid < rem)
        def _():
            pltpu.sync_copy(src_hbm.at[pl.ds(cw + bpt, 1)],
                            src_v.at[pl.ds(bpt, 1)])
            pltpu.sync_copy(dst_hbm.at[pl.ds(cw + bpt, 1)],
                            dst_v.at[pl.ds(bpt, 1)])
        plsc.subcore_barrier()

        def drain(b):
            pltpu.make_async_copy(g_hbm.at[pl.ds(0, CB)], bufs[b],
                                  sems[b]).wait()

        def emit_visit(j_val, b, bp, with_reuse_wait):
            if with_reuse_wait:
                drain(b)
            pltpu.async_copy(g_sh.at[src_v.at[j_val]], bufs[b], sems[b])
            drain(bp)
            pltpu.async_copy(bufs[bp], acc_sh.at[dst_v.at[j_val - 1]],
                             sems[bp], add=True)

        pltpu.async_copy(g_sh.at[src_v.at[0]], bufs[0], sems[0])
        peel_extra = (bpt - NBUF) % NBUF
        n_rounds = (bpt - NBUF - peel_extra) // NBUF
        for j in range(1, NBUF + peel_extra):
            emit_visit(j, j % NBUF, (j - 1) % NBUF, j >= NBUF)

        j0 = NBUF + peel_extra

        def round_body(g, _):
            base = j0 + g * NBUF
            for u in range(NBUF):
                b = (j0 + u) % NBUF
                bp = (j0 + u - 1) % NBUF
                emit_visit(base + u, b, bp, True)
            return 0
        lax.fori_loop(0, n_rounds, round_body, 0)
        drain((bpt - 1) % NBUF)
        pltpu.async_copy(bufs[(bpt - 1) % NBUF],
                         acc_sh.at[dst_v.at[bpt - 1]],
                         sems[(bpt - 1) % NBUF], add=True)

        for b in range(NBUF):
            drain(b)

        @pl.when(wid < rem)
        def _():
            pltpu.async_copy(g_sh.at[src_v.at[bpt]], bufs[0], sems[0]).wait()
            pltpu.async_copy(bufs[0], acc_sh.at[dst_v.at[bpt]],
                             sems[0], add=True)
            drain(0)
        plsc.subcore_barrier()
        pltpu.sync_copy(acc_sh.at[pl.ds(s * RPT, RPT)],
                        out_hbm.at[c, pl.ds(s * RPT, RPT)])

    return agg_kernel


def _dense1_body(deg_ref, x_ref, w1_ref, dinv_ref, g1_ref):
    deg = deg_ref[0] + deg_ref[1]
    dinv = lax.rsqrt(deg + 1.0)
    h = jnp.dot(x_ref[...], w1_ref[...], preferred_element_type=jnp.float32)
    dinv_ref[...] = dinv
    g1_ref[pl.ds(0, N), :] = dinv[:N, :] * h
    g1_ref[pl.ds(N, NPAD - N), :] = jnp.zeros((NPAD - N, HID), jnp.float32)


def _dense2_body(a_ref, g_ref, dinv_ref, b_ref, w2_ref, g2_ref):
    acc = a_ref[0] + a_ref[1] + g_ref[...]
    h1 = jnp.maximum(dinv_ref[...] * acc + b_ref[...], 0.0)
    g2_ref[...] = dinv_ref[...] * jnp.dot(
        h1, w2_ref[...], preferred_element_type=jnp.float32)


def _final_body(a_ref, g_ref, dinv_ref, b_ref, wfc_ref, bfc_ref, out_ref):
    acc = a_ref[0] + a_ref[1] + g_ref[...]
    h2 = jnp.maximum(dinv_ref[...] * acc + b_ref[...], 0.0)
    mask = lax.broadcasted_iota(jnp.int32, (NPAD, HID), 0) < N
    h2 = jnp.where(mask, h2, 0.0)
    pooled = jnp.sum(h2, axis=0, keepdims=True) * (1.0 / N)
    y = jnp.sum(pooled * wfc_ref[...], axis=1, keepdims=True) + bfc_ref[...]
    out_ref[...] = 1.0 / (1.0 + jnp.exp(-y))


def kernel(x, edge_index, W1, b1, W2, b2, Wfc, bfc):
    E = edge_index.shape[1]
    tch = E // CB
    src2d = edge_index[0].astype(jnp.int32).reshape(tch, CB)
    dst2d = edge_index[1].astype(jnp.int32).reshape(tch, CB)
    deg_parts = _make_deg_kernel(tch)(dst2d)

    dinv1, g1 = pl.pallas_call(
        _dense1_body,
        out_shape=[jax.ShapeDtypeStruct((NPAD, HID), jnp.float32),
                   jax.ShapeDtypeStruct((NPAD, HID), jnp.float32)],
    )(deg_parts, x, W1)

    agg = _make_agg_kernel(tch)
    a1 = agg(g1, src2d, dst2d)

    g2 = pl.pallas_call(
        _dense2_body,
        out_shape=jax.ShapeDtypeStruct((NPAD, HID), jnp.float32),
    )(a1, g1, dinv1, b1.reshape(1, HID), W2)

    a2 = agg(g2, src2d, dst2d)

    y = pl.pallas_call(
        _final_body,
        out_shape=jax.ShapeDtypeStruct((1, 1), jnp.float32),
    )(a2, g2, dinv1, b2.reshape(1, HID), Wfc.reshape(1, HID),
      bfc.reshape(1, 1))
    return y.reshape(1)


# speedup vs baseline: 1.0614x; 1.0011x over previous
import functools

import jax
import jax.numpy as jnp
from jax import lax
from jax.experimental import pallas as pl
from jax.experimental.pallas import tpu as pltpu
from jax.experimental.pallas import tpu_sc as plsc

N = 10000
HID = 16
NC, NS = 2, 16
NW = NC * NS
CB = 128
NPAD = 10240
RPT = NPAD // NS
NBUF = 4

_mesh = plsc.VectorSubcoreMesh(core_axis_name="c", subcore_axis_name="s")


def _fill(buf, val, rows):
    def body(i, _):
        buf[i, :] = jnp.full((HID,), val, jnp.float32)
        return 0
    lax.fori_loop(0, rows, body, 0)


def _chunk_split(tch):
    return tch // NW, tch % NW


def _make_deg_kernel(tch):
    bpt, rem = _chunk_split(tch)

    @functools.partial(
        pl.kernel,
        mesh=_mesh,
        out_type=jax.ShapeDtypeStruct((NC, NPAD, HID), jnp.float32),
        scratch_types=[
            pltpu.VMEM((bpt + 1, CB), jnp.int32),
            pltpu.VMEM((CB, HID), jnp.float32),
            pltpu.VMEM_SHARED((NPAD, HID), jnp.float32),
            pltpu.SemaphoreType.DMA,
        ],
        compiler_params=pltpu.CompilerParams(use_tc_tiling_on_sc=False),
    )
    def deg_kernel(dst_hbm, out_hbm, dst_v, buf_v, acc_sh, sem):
        c = lax.axis_index("c")
        s = lax.axis_index("s")
        wid = c * NS + s
        cw = bpt * wid + jnp.minimum(wid, rem)
        _fill(buf_v, 0.0, CB)
        for k in range(RPT // CB):
            pltpu.sync_copy(buf_v, acc_sh.at[pl.ds(s * RPT + k * CB, CB)])
        _fill(buf_v, 1.0, CB)
        pltpu.sync_copy(dst_hbm.at[pl.ds(cw, bpt)],
                        dst_v.at[pl.ds(0, bpt)])

        @pl.when(wid < rem)
        def _():
            pltpu.sync_copy(dst_hbm.at[pl.ds(cw + bpt, 1)],
                            dst_v.at[pl.ds(bpt, 1)])
        plsc.subcore_barrier()

        depth = 6
        for j in range(depth):
            pltpu.async_copy(buf_v, acc_sh.at[dst_v.at[j]], sem, add=True)

        def body(j, _):
            pltpu.make_async_copy(out_hbm.at[0, pl.ds(0, CB)], buf_v,
                                  sem).wait()
            pltpu.async_copy(buf_v, acc_sh.at[dst_v.at[j + depth]], sem,
                             add=True)
            return 0
        lax.fori_loop(0, bpt - depth, body, 0)

        @pl.when(wid < rem)
        def _():
            pltpu.async_copy(buf_v, acc_sh.at[dst_v.at[bpt]], sem, add=True)
        for _ in range(depth):
            pltpu.make_async_copy(out_hbm.at[0, pl.ds(0, CB)], buf_v,
                                  sem).wait()

        @pl.when(wid < rem)
        def _():
            pltpu.make_async_copy(out_hbm.at[0, pl.ds(0, CB)], buf_v,
                                  sem).wait()
        plsc.subcore_barrier()
        pltpu.sync_copy(acc_sh.at[pl.ds(s * RPT, RPT)],
                        out_hbm.at[c, pl.ds(s * RPT, RPT)])

    return deg_kernel


def _make_agg_kernel(tch):
    bpt, rem = _chunk_split(tch)

    @functools.partial(
        pl.kernel,
        mesh=_mesh,
        out_type=jax.ShapeDtypeStruct((NC, NPAD, HID), jnp.float32),
        scratch_types=[
            pltpu.VMEM((bpt + 1, CB), jnp.int32),
            pltpu.VMEM((bpt + 1, CB), jnp.int32),
            [pltpu.VMEM((CB, HID), jnp.float32)] * NBUF,
            pltpu.VMEM_SHARED((NPAD, HID), jnp.float32),
            pltpu.VMEM_SHARED((NPAD, HID), jnp.float32),
            [pltpu.SemaphoreType.DMA] * NBUF,
        ],
        compiler_params=pltpu.CompilerParams(use_tc_tiling_on_sc=False),
    )
    def agg_kernel(g_hbm, src_hbm, dst_hbm, out_hbm,
                   src_v, dst_v, bufs, acc_sh, g_sh, sems):
        c = lax.axis_index("c")
        s = lax.axis_index("s")
        wid = c * NS + s
        cw = bpt * wid + jnp.minimum(wid, rem)
        _fill(bufs[0], 0.0, CB)
        for k in range(RPT // CB):
            pltpu.sync_copy(bufs[0], acc_sh.at[pl.ds(s * RPT + k * CB, CB)])
        pltpu.sync_copy(g_hbm.at[pl.ds(s * RPT, RPT)],
                        g_sh.at[pl.ds(s * RPT, RPT)])
        pltpu.sync_copy(src_hbm.at[pl.ds(cw, bpt)], src_v.at[pl.ds(0, bpt)])
        pltpu.sync_copy(dst_hbm.at[pl.ds(cw, bpt)], dst_v.at[pl.ds(0, bpt)])

        @pl.when(wid < rem)
        def _():
            pltpu.sync_copy(src_hbm.at[pl.ds(cw + bpt, 1)],
                            src_v.at[pl.ds(bpt, 1)])
            pltpu.sync_copy(dst_hbm.at[pl.ds(cw + bpt, 1)],
                            dst_v.at[pl.ds(bpt, 1)])
        plsc.subcore_barrier()

        def drain(b):
            pltpu.make_async_copy(g_hbm.at[pl.ds(0, CB)], bufs[b],
                                  sems[b]).wait()

        def emit_visit(j_val, b, bp, with_reuse_wait):
            if with_reuse_wait:
                drain(b)
            pltpu.async_copy(g_sh.at[src_v.at[j_val]], bufs[b], sems[b])
            drain(bp)
            pltpu.async_copy(bufs[bp], acc_sh.at[dst_v.at[j_val - 1]],
                             sems[bp], add=True)

        pltpu.async_copy(g_sh.at[src_v.at[0]], bufs[0], sems[0])
        peel_extra = (bpt - NBUF) % NBUF
        n_rounds = (bpt - NBUF - peel_extra) // NBUF
        for j in range(1, NBUF + peel_extra):
            emit_visit(j, j % NBUF, (j - 1) % NBUF, j >= NBUF)

        j0 = NBUF + peel_extra

        def round_body(g, _):
            base = j0 + g * NBUF
            for u in range(NBUF):
                b = (j0 + u) % NBUF
                bp = (j0 + u - 1) % NBUF
                emit_visit(base + u, b, bp, True)
            return 0
        lax.fori_loop(0, n_rounds, round_body, 0)
        drain((bpt - 1) % NBUF)
        pltpu.async_copy(bufs[(bpt - 1) % NBUF],
                         acc_sh.at[dst_v.at[bpt - 1]],
                         sems[(bpt - 1) % NBUF], add=True)

        for b in range(NBUF):
            drain(b)

        @pl.when(wid < rem)
        def _():
            pltpu.async_copy(g_sh.at[src_v.at[bpt]], bufs[0], sems[0]).wait()
            pltpu.async_copy(bufs[0], acc_sh.at[dst_v.at[bpt]],
                             sems[0], add=True)
            drain(0)
        plsc.subcore_barrier()
        pltpu.sync_copy(acc_sh.at[pl.ds(s * RPT, RPT)],
                        out_hbm.at[c, pl.ds(s * RPT, RPT)])

    return agg_kernel


def _mm_body(x_ref, w1_ref, h_ref):
    h = jnp.dot(x_ref[...], w1_ref[...], preferred_element_type=jnp.float32)
    h_ref[pl.ds(0, N), :] = h
    h_ref[pl.ds(N, NPAD - N), :] = jnp.zeros((NPAD - N, HID), jnp.float32)


def _scale_body(deg_ref, h_ref, dinv_ref, g1_ref):
    deg = deg_ref[0] + deg_ref[1]
    dinv = lax.rsqrt(deg + 1.0)
    dinv_ref[...] = dinv
    g1_ref[...] = dinv * h_ref[...]


def _dense2_body(a_ref, g_ref, dinv_ref, b_ref, w2_ref, g2_ref):
    acc = a_ref[0] + a_ref[1] + g_ref[...]
    h1 = jnp.maximum(dinv_ref[...] * acc + b_ref[...], 0.0)
    g2_ref[...] = dinv_ref[...] * jnp.dot(
        h1, w2_ref[...], preferred_element_type=jnp.float32)


def _final_body(a_ref, g_ref, dinv_ref, b_ref, wfc_ref, bfc_ref, out_ref):
    acc = a_ref[0] + a_ref[1] + g_ref[...]
    h2 = jnp.maximum(dinv_ref[...] * acc + b_ref[...], 0.0)
    mask = lax.broadcasted_iota(jnp.int32, (NPAD, HID), 0) < N
    h2 = jnp.where(mask, h2, 0.0)
    pooled = jnp.sum(h2, axis=0, keepdims=True) * (1.0 / N)
    y = jnp.sum(pooled * wfc_ref[...], axis=1, keepdims=True) + bfc_ref[...]
    out_ref[...] = 1.0 / (1.0 + jnp.exp(-y))


def kernel(x, edge_index, W1, b1, W2, b2, Wfc, bfc):
    E = edge_index.shape[1]
    tch = E // CB
    src2d = edge_index[0].astype(jnp.int32).reshape(tch, CB)
    dst2d = edge_index[1].astype(jnp.int32).reshape(tch, CB)
    h1raw = pl.pallas_call(
        _mm_body,
        out_shape=jax.ShapeDtypeStruct((NPAD, HID), jnp.float32),
    )(x, W1)
    deg_parts = _make_deg_kernel(tch)(dst2d)

    dinv1, g1 = pl.pallas_call(
        _scale_body,
        out_shape=[jax.ShapeDtypeStruct((NPAD, HID), jnp.float32),
                   jax.ShapeDtypeStruct((NPAD, HID), jnp.float32)],
    )(deg_parts, h1raw)

    agg = _make_agg_kernel(tch)
    a1 = agg(g1, src2d, dst2d)

    g2 = pl.pallas_call(
        _dense2_body,
        out_shape=jax.ShapeDtypeStruct((NPAD, HID), jnp.float32),
    )(a1, g1, dinv1, b1.reshape(1, HID), W2)

    a2 = agg(g2, src2d, dst2d)

    y = pl.pallas_call(
        _final_body,
        out_shape=jax.ShapeDtypeStruct((1, 1), jnp.float32),
    )(a2, g2, dinv1, b2.reshape(1, HID), Wfc.reshape(1, HID),
      bfc.reshape(1, 1))
    return y.reshape(1)
